# Initial kernel scaffold; baseline (speedup 1.0000x reference)
import jax
import jax.numpy as jnp
from jax.experimental import pallas as pl


def kernel(x, edge_index, pred_edge_index, Wl1, bl1, Wr1, g1, be1, Wl2, bl2, Wr2, W1, c1, pg1, pbe1, W2, c2, pg2, pbe2, W3, c3):
    raise NotImplementedError("write your pallas kernel here")


# speedup vs baseline: 3.7174x; 3.7174x over previous
import jax
import jax.numpy as jnp
from jax import lax
from jax.experimental import pallas as pl
from jax.experimental.pallas import tpu as pltpu
from jax.experimental.pallas import tpu_sc as plsc

N = 10000
E = 320000
DIN = 128
DH = 128
DOUT = 64
EPS = 1e-5
BN_SCALE = 1.0 / (1.0 + EPS) ** 0.5

NC = 2
NS = 16
NW = NC * NS
CH = 80
EPW = E // NW
NCHUNK = EPW // CH
SLAB = 624
SLAB_LAST = N - (NS - 1) * SLAB

_f32 = jnp.float32


def _sc_mesh():
    return plsc.VectorSubcoreMesh(
        core_axis_name="c", subcore_axis_name="s", num_cores=NC, num_subcores=NS
    )


def _make_segsum(with_counts):
    out_type = [jax.ShapeDtypeStruct((NC, N, DH), _f32)]
    scratch = [
        pltpu.VMEM((NCHUNK, CH), jnp.int32),
        pltpu.VMEM((NCHUNK, CH), jnp.int32),
        pltpu.VMEM((CH, DH), _f32),
        pltpu.VMEM_SHARED((N, DH), _f32),
        pltpu.SemaphoreType.DMA,
    ]
    if with_counts:
        out_type.append(jax.ShapeDtypeStruct((NC * N,), _f32))
        scratch += [
            pltpu.VMEM((CH,), _f32),
            pltpu.VMEM_SHARED((N,), _f32),
            pltpu.VMEM((N // 2,), _f32),
        ]

    def body(src_hbm, dst_hbm, tbl_hbm, zrows_hbm, zvec_hbm, s_out, *rest):
        if with_counts:
            cnt_out, idx_s, idx_d, rows, accum_sh, sem, ones, cnt_sh, stage = rest
        else:
            cnt_out, ones, cnt_sh, stage = None, None, None, None
            idx_s, idx_d, rows, accum_sh, sem = rest
        c = lax.axis_index("c")
        s = lax.axis_index("s")
        wid = c * NS + s
        slab0 = s * SLAB

        @pl.when(s < NS - 1)
        def _():
            pltpu.sync_copy(zrows_hbm.at[pl.ds(slab0, SLAB)],
                            accum_sh.at[pl.ds(slab0, SLAB)])

        @pl.when(s == NS - 1)
        def _():
            pltpu.sync_copy(zrows_hbm.at[pl.ds((NS - 1) * SLAB, SLAB_LAST)],
                            accum_sh.at[pl.ds((NS - 1) * SLAB, SLAB_LAST)])

        if with_counts:
            @pl.when(s < 2)
            def _():
                pltpu.sync_copy(zvec_hbm.at[pl.ds(s * (N // 2), N // 2)], stage)
                pltpu.sync_copy(stage, cnt_sh.at[pl.ds(s * (N // 2), N // 2)])
            for i in range(CH // 16):
                ones[pl.ds(i * 16, 16)] = jnp.ones((16,), _f32)

        pltpu.sync_copy(src_hbm.at[wid], idx_s)
        pltpu.sync_copy(dst_hbm.at[wid], idx_d)
        plsc.subcore_barrier()

        def chunk(j, carry):
            pltpu.async_copy(tbl_hbm.at[idx_s.at[j]], rows, sem).wait()
            pltpu.sync_copy(rows, accum_sh.at[idx_d.at[j]], add=True)
            if with_counts:
                pltpu.sync_copy(ones, cnt_sh.at[idx_d.at[j]], add=True)
            return carry

        lax.fori_loop(0, NCHUNK, chunk, 0)
        plsc.subcore_barrier()

        @pl.when(s < NS - 1)
        def _():
            pltpu.sync_copy(accum_sh.at[pl.ds(slab0, SLAB)],
                            s_out.at[c, pl.ds(slab0, SLAB)])

        @pl.when(s == NS - 1)
        def _():
            pltpu.sync_copy(accum_sh.at[pl.ds((NS - 1) * SLAB, SLAB_LAST)],
                            s_out.at[c, pl.ds((NS - 1) * SLAB, SLAB_LAST)])

        if with_counts:
            @pl.when(s < 2)
            def _():
                pltpu.sync_copy(cnt_sh.at[pl.ds(s * (N // 2), N // 2)], stage)
                pltpu.sync_copy(stage,
                                cnt_out.at[pl.ds(c * N + s * (N // 2), N // 2)])

    return pl.kernel(body, out_type=out_type, mesh=_sc_mesh(),
                     scratch_types=scratch)


def _make_pred_gather():
    out_type = [jax.ShapeDtypeStruct((E, DH), _f32),
                jax.ShapeDtypeStruct((E, DH), _f32)]
    scratch = [
        pltpu.VMEM((NCHUNK, CH), jnp.int32),
        pltpu.VMEM((NCHUNK, CH), jnp.int32),
        pltpu.VMEM((CH, DH), _f32),
        pltpu.VMEM((CH, DH), _f32),
        pltpu.SemaphoreType.DMA,
        pltpu.SemaphoreType.DMA,
    ]

    def body(ps_hbm, pd_hbm, u_hbm, v_hbm, tu_out, tv_out,
             idx_s, idx_d, bufs, bufd, sem0, sem1):
        c = lax.axis_index("c")
        s = lax.axis_index("s")
        wid = c * NS + s
        pltpu.sync_copy(ps_hbm.at[wid], idx_s)
        pltpu.sync_copy(pd_hbm.at[wid], idx_d)

        def chunk(j, carry):
            a = pltpu.async_copy(u_hbm.at[idx_s.at[j]], bufs, sem0)
            b = pltpu.async_copy(v_hbm.at[idx_d.at[j]], bufd, sem1)
            a.wait()
            b.wait()
            base = wid * EPW + j * CH
            pltpu.sync_copy(bufs, tu_out.at[pl.ds(base, CH)])
            pltpu.sync_copy(bufd, tv_out.at[pl.ds(base, CH)])
            return carry

        lax.fori_loop(0, NCHUNK, chunk, 0)

    return pl.kernel(body, out_type=out_type, mesh=_sc_mesh(),
                     scratch_types=scratch)


_NB = 1000


def _tc_b_body(s0, s1, c0, c1, x, wl1, wr1, bl1, g1, be1, wr2, h_out, hw2):
    cnt = jnp.maximum(c0[...] + c1[...], 1.0)
    aggr = (s0[...] + s1[...]) / cnt
    pre = (jnp.dot(aggr, wl1[...], preferred_element_type=_f32) + bl1[...]
           + jnp.dot(x[...], wr1[...], preferred_element_type=_f32))
    h = jnp.maximum(pre * (g1[...] * BN_SCALE) + be1[...], 0.0)
    h_out[...] = h
    hw2[...] = jnp.dot(h, wr2[...], preferred_element_type=_f32)


def _tc_b(s1p0, s1p1, cnt0, cnt1, x, wl1, wr1, bl1, g1, be1, wr2):
    row = lambda i: (i, 0)
    fix = lambda i: (0, 0)
    return pl.pallas_call(
        _tc_b_body,
        grid=(N // _NB,),
        in_specs=[
            pl.BlockSpec((_NB, DH), row),
            pl.BlockSpec((_NB, DH), row),
            pl.BlockSpec((_NB, 1), row),
            pl.BlockSpec((_NB, 1), row),
            pl.BlockSpec((_NB, DIN), row),
            pl.BlockSpec((DIN, DH), fix),
            pl.BlockSpec((DIN, DH), fix),
            pl.BlockSpec((1, DH), fix),
            pl.BlockSpec((1, DH), fix),
            pl.BlockSpec((1, DH), fix),
            pl.BlockSpec((DH, DOUT), fix),
        ],
        out_specs=[pl.BlockSpec((_NB, DH), row),
                   pl.BlockSpec((_NB, DOUT), row)],
        out_shape=[jax.ShapeDtypeStruct((N, DH), _f32),
                   jax.ShapeDtypeStruct((N, DOUT), _f32)],
    )(s1p0, s1p1, cnt0, cnt1, x, wl1, wr1, bl1, g1, be1, wr2)


def _tc_c_body(s0, s1, c0, c1, hw2, wl2, bl2, w1, c1b, u_out, v_out):
    cnt = jnp.maximum(c0[...] + c1[...], 1.0)
    aggr = (s0[...] + s1[...]) / cnt
    z = (jnp.dot(aggr, wl2[...], preferred_element_type=_f32) + bl2[...]
         + hw2[...])
    w1m = w1[...]
    u_out[...] = jnp.dot(z, w1m[:DOUT], preferred_element_type=_f32) + c1b[...]
    v_out[...] = jnp.dot(z, w1m[DOUT:], preferred_element_type=_f32)


def _tc_c(s2p0, s2p1, cnt0, cnt1, hw2, wl2, bl2, w1, c1b):
    row = lambda i: (i, 0)
    fix = lambda i: (0, 0)
    return pl.pallas_call(
        _tc_c_body,
        grid=(N // _NB,),
        in_specs=[
            pl.BlockSpec((_NB, DH), row),
            pl.BlockSpec((_NB, DH), row),
            pl.BlockSpec((_NB, 1), row),
            pl.BlockSpec((_NB, 1), row),
            pl.BlockSpec((_NB, DOUT), row),
            pl.BlockSpec((DH, DOUT), fix),
            pl.BlockSpec((1, DOUT), fix),
            pl.BlockSpec((2 * DOUT, DH), fix),
            pl.BlockSpec((1, DH), fix),
        ],
        out_specs=[pl.BlockSpec((_NB, DH), row),
                   pl.BlockSpec((_NB, DH), row)],
        out_shape=[jax.ShapeDtypeStruct((N, DH), _f32),
                   jax.ShapeDtypeStruct((N, DH), _f32)],
    )(s2p0, s2p1, cnt0, cnt1, hw2, wl2, bl2, w1, c1b)


_EB = 512


def _tc_d_body(tu, tv, pg1, pbe1, w2, c2, pg2, pbe2, w3, c3, out):
    t = tu[...] + tv[...]
    t = jnp.maximum(t * (pg1[...] * BN_SCALE) + pbe1[...], 0.0)
    t = jnp.dot(t, w2[...], preferred_element_type=_f32) + c2[...]
    t = jnp.maximum(t * (pg2[...] * BN_SCALE) + pbe2[...], 0.0)
    out[...] = jnp.sum(t * w3[...], axis=1) + c3[0]


def _tc_d(tu, tv, pg1, pbe1, w2, c2, pg2, pbe2, w3row, c3):
    row = lambda i: (i, 0)
    fix = lambda i: (0, 0)
    return pl.pallas_call(
        _tc_d_body,
        grid=(E // _EB,),
        in_specs=[
            pl.BlockSpec((_EB, DH), row),
            pl.BlockSpec((_EB, DH), row),
            pl.BlockSpec((1, DH), fix),
            pl.BlockSpec((1, DH), fix),
            pl.BlockSpec((DH, DOUT), fix),
            pl.BlockSpec((1, DOUT), fix),
            pl.BlockSpec((1, DOUT), fix),
            pl.BlockSpec((1, DOUT), fix),
            pl.BlockSpec((1, DOUT), fix),
            pl.BlockSpec(memory_space=pltpu.SMEM),
        ],
        out_specs=pl.BlockSpec((_EB,), lambda i: (i,)),
        out_shape=jax.ShapeDtypeStruct((E,), _f32),
    )(tu, tv, pg1, pbe1, w2, c2, pg2, pbe2, w3row, c3)


def kernel(x, edge_index, pred_edge_index, Wl1, bl1, Wr1, g1, be1, Wl2, bl2,
           Wr2, W1, c1, pg1, pbe1, W2, c2, pg2, pbe2, W3, c3):
    src3d = edge_index[0].reshape(NW, NCHUNK, CH)
    dst3d = edge_index[1].reshape(NW, NCHUNK, CH)
    ps3d = pred_edge_index[0].reshape(NW, NCHUNK, CH)
    pd3d = pred_edge_index[1].reshape(NW, NCHUNK, CH)
    zrows = jnp.zeros((N, DH), _f32)
    zvec = jnp.zeros((N,), _f32)

    s1p, cntp = _make_segsum(True)(src3d, dst3d, x, zrows, zvec)
    cnt0 = cntp[:N].reshape(N, 1)
    cnt1 = cntp[N:].reshape(N, 1)

    h, hw2 = _tc_b(s1p[0], s1p[1], cnt0, cnt1, x, Wl1, Wr1,
                   bl1.reshape(1, DH), g1.reshape(1, DH), be1.reshape(1, DH),
                   Wr2)

    (s2p,) = _make_segsum(False)(src3d, dst3d, h, zrows, zvec)

    u, v = _tc_c(s2p[0], s2p[1], cnt0, cnt1, hw2, Wl2,
                 bl2.reshape(1, DOUT), W1, c1.reshape(1, DH))

    tu, tv = _make_pred_gather()(ps3d, pd3d, u, v)

    return _tc_d(tu, tv,
                 pg1.reshape(1, DH), pbe1.reshape(1, DH),
                 W2, c2.reshape(1, DOUT), pg2.reshape(1, DOUT),
                 pbe2.reshape(1, DOUT), W3.reshape(1, DOUT), c3)
